# baseline (speedup 1.0000x reference)
import functools

import jax
import jax.numpy as jnp
from jax import lax
from jax.experimental import pallas as pl
from jax.experimental.pallas import tpu as pltpu
from jax.experimental.pallas import tpu_sc as plsc

HIDDEN = 1024
FFN = 4096
E = 8
EP = 128
T = 2048
NPAIR = 2 * T
BM = 128
P = NPAIR + E * BM
NB = P // BM
BF = 512
NF = FFN // BF
NW = 32
CPW = NPAIR // NW
L = 16
NG = NPAIR // L


def _router_body(x_ref, wr_ref, idx_ref, w_ref, base_ref, bexp_ref, bnr_ref):
    x = x_ref[...]
    wr = wr_ref[...]
    logits = lax.dot_general(x, wr, (((1,), (1,)), ((), ())),
                             preferred_element_type=jnp.float32)
    col = lax.broadcasted_iota(jnp.int32, logits.shape, 1)
    neg = jnp.float32(-1e30)
    logits = jnp.where(col < E, logits, neg)
    m1 = jnp.max(logits, axis=1, keepdims=True)
    a1 = jnp.min(jnp.where(logits == m1, col, EP), axis=1, keepdims=True)
    l2 = jnp.where(col == a1, neg, logits)
    m2 = jnp.max(l2, axis=1, keepdims=True)
    a2 = jnp.min(jnp.where(l2 == m2, col, EP), axis=1, keepdims=True)
    wtop = 1.0 / (1.0 + jnp.exp(m2 - m1))
    idx_ref[...] = jnp.where(col == 0, a1, 0) + jnp.where(col == 1, a2, 0)
    w_ref[...] = (jnp.where(col == 0, wtop, 0.0)
                  + jnp.where(col == 1, 1.0 - wtop, 0.0))

    onehot = (jnp.where(col == a1, 1.0, 0.0)
              + jnp.where(col == a2, 1.0, 0.0))
    trow = lax.broadcasted_iota(jnp.int32, (NG, T), 1)
    wrow = lax.broadcasted_iota(jnp.int32, (NG, T), 0)
    seg = jnp.where(trow // (T // NG) == wrow, 1.0, 0.0)
    hist = jnp.dot(seg, onehot, preferred_element_type=jnp.float32)
    total = jnp.sum(hist, axis=0, keepdims=True)
    padded = jnp.floor((total + (BM - 1)) / BM) * BM
    li = lax.broadcasted_iota(jnp.int32, (EP, EP), 0)
    lj = lax.broadcasted_iota(jnp.int32, (EP, EP), 1)
    lower = jnp.where(li < lj, 1.0, 0.0)
    gstart = jnp.dot(padded, lower, preferred_element_type=jnp.float32)
    wi = lax.broadcasted_iota(jnp.int32, (NG, NG), 0)
    wj = lax.broadcasted_iota(jnp.int32, (NG, NG), 1)
    wlow = jnp.where(wj < wi, 1.0, 0.0)
    prefix = jnp.dot(wlow, hist, preferred_element_type=jnp.float32)
    base_ref[...] = (gstart + prefix).astype(jnp.int32)

    bs = lax.broadcasted_iota(jnp.int32, (NB, EP), 0).astype(jnp.float32) * BM
    ecol = lax.broadcasted_iota(jnp.int32, (NB, EP), 1).astype(jnp.float32)
    gs = jnp.broadcast_to(gstart, (NB, EP))
    pd = jnp.broadcast_to(padded, (NB, EP))
    tt = jnp.broadcast_to(total, (NB, EP))
    inb = jnp.where((bs >= gs) & (bs < gs + pd), 1.0, 0.0)
    anyb = jnp.sum(inb, axis=1, keepdims=True)
    laste = jnp.max(jnp.where(total > 0, ecol[:1], 0.0), axis=1, keepdims=True)
    bexp = jnp.sum(inb * ecol, axis=1, keepdims=True) + (1.0 - anyb) * laste
    nr = jnp.clip(tt - (bs - gs), 0.0, float(BM))
    bnr = jnp.sum(inb * nr, axis=1, keepdims=True)
    ccol = lax.broadcasted_iota(jnp.int32, (NB, EP), 1)
    bexp_ref[...] = jnp.where(ccol == 0, bexp.astype(jnp.int32), 0)
    bnr_ref[...] = jnp.where(ccol == 0, bnr.astype(jnp.int32), 0)


def _router(xf, wr):
    return pl.pallas_call(
        _router_body,
        out_shape=(
            jax.ShapeDtypeStruct((T, EP), jnp.int32),
            jax.ShapeDtypeStruct((T, EP), jnp.float32),
            jax.ShapeDtypeStruct((NG, EP), jnp.int32),
            jax.ShapeDtypeStruct((NB, EP), jnp.int32),
            jax.ShapeDtypeStruct((NB, EP), jnp.int32),
        ),
    )(xf, wr)


def _iota16():
    return lax.iota(jnp.int32, L)


def _take16(v, idx):
    return lax.gather(
        v, idx[:, None],
        lax.GatherDimensionNumbers(offset_dims=(), collapsed_slice_dims=(0,),
                                   start_index_map=(0,)),
        slice_sizes=(1,),
        mode=lax.GatherScatterMode.PROMISE_IN_BOUNDS)


def _dispatch_body(epair_hbm, base_hbm, x_hbm, xs_hbm, pos_hbm,
                   eids_v, base_v, posb_v, tokb_v, rows_v, sem_g):
    wid = lax.axis_index("s") * 2 + lax.axis_index("c")
    p0 = wid * CPW
    g0 = pl.multiple_of(p0 // L, CPW // L)
    pltpu.sync_copy(epair_hbm.at[pl.ds(p0, CPW)], eids_v)
    pltpu.sync_copy(base_hbm.at[pl.ds(g0, CPW // L)], base_v)
    iot = _iota16()
    for j in range(CPW // L):
        base_j = base_v[j]
        v = eids_v[pl.ds(j * L, L)]
        bp = _take16(base_j, v)
        rank = jnp.zeros((L,), jnp.int32)
        for k in range(1, L):
            vk = _take16(v, jnp.maximum(iot - k, 0))
            rank = rank + jnp.where((iot >= k) & (vk == v), 1, 0)
        posb_v[j // 4, pl.ds((j % 4) * L, L)] = bp + rank
        tokb_v[j // 4, pl.ds((j % 4) * L, L)] = (p0 + j * L + iot) >> 1
    for c in range(2):
        pltpu.sync_copy(posb_v.at[c], pos_hbm.at[pl.ds(p0 + c * 64, 64)])
        pltpu.async_copy(x_hbm.at[tokb_v.at[c]], rows_v, sem_g).wait()
        pltpu.sync_copy(rows_v, xs_hbm.at[posb_v.at[c]])


def _dispatch(e_pair, base, xf):
    kfn = pl.kernel(
        _dispatch_body,
        out_type=(
            jax.ShapeDtypeStruct((P, HIDDEN), jnp.float32),
            jax.ShapeDtypeStruct((NPAIR,), jnp.int32),
        ),
        mesh=plsc.VectorSubcoreMesh(core_axis_name="c", subcore_axis_name="s"),
        scratch_types=(
            pltpu.VMEM((CPW,), jnp.int32),
            pltpu.VMEM((CPW // L, L), jnp.int32),
            pltpu.VMEM((2, 64), jnp.int32),
            pltpu.VMEM((2, 64), jnp.int32),
            pltpu.VMEM((64, HIDDEN), jnp.float32),
            pltpu.SemaphoreType.DMA,
        ),
    )
    return kfn(e_pair, base, xf)


def _gmm_body(bexp_ref, bnr_ref, xs_ref, w1_ref, w3_ref, w2_ref, ys_ref):
    f = pl.program_id(0)
    b = pl.program_id(1)

    @pl.when(bnr_ref[b] > 0)
    def _():
        x = xs_ref[pl.ds(b * BM, BM), :]
        h1 = jnp.dot(x, w1_ref[0], preferred_element_type=jnp.float32)
        h3 = jnp.dot(x, w3_ref[0], preferred_element_type=jnp.float32)
        h = (h1 * jax.nn.sigmoid(h1)) * h3
        delta = jnp.dot(h, w2_ref[0], preferred_element_type=jnp.float32)

        @pl.when(f == 0)
        def _():
            ys_ref[pl.ds(b * BM, BM), :] = delta

        @pl.when(f != 0)
        def _():
            ys_ref[pl.ds(b * BM, BM), :] += delta


def _gmm(bexp, bnr, xs, w1, w3, w2):
    grid_spec = pltpu.PrefetchScalarGridSpec(
        num_scalar_prefetch=2,
        grid=(NF, NB),
        in_specs=[
            pl.BlockSpec((P, HIDDEN), lambda f, b, be, bn: (0, 0)),
            pl.BlockSpec((1, HIDDEN, BF), lambda f, b, be, bn: (be[b], 0, f)),
            pl.BlockSpec((1, HIDDEN, BF), lambda f, b, be, bn: (be[b], 0, f)),
            pl.BlockSpec((1, BF, HIDDEN), lambda f, b, be, bn: (be[b], f, 0)),
        ],
        out_specs=pl.BlockSpec((P, HIDDEN), lambda f, b, be, bn: (0, 0)),
    )
    return pl.pallas_call(
        _gmm_body,
        grid_spec=grid_spec,
        out_shape=jax.ShapeDtypeStruct((P, HIDDEN), jnp.float32),
        compiler_params=pltpu.CompilerParams(
            dimension_semantics=("arbitrary", "arbitrary"),
        ),
    )(bexp, bnr, xs, w1, w3, w2)


TPC = 8


def _combine_body(ys_hbm, pos_hbm, wp_hbm, out_hbm,
                  posb_v, wb_v, rows_v, outb_v, sem_g):
    wid = lax.axis_index("s") * 2 + lax.axis_index("c")
    p0 = wid * CPW
    pltpu.sync_copy(pos_hbm.at[pl.ds(p0, CPW)], posb_v)
    pltpu.sync_copy(wp_hbm.at[pl.ds(p0, CPW)], wb_v)
    nch = CPW // (2 * TPC)
    for c in range(nch):
        pltpu.async_copy(ys_hbm.at[posb_v.at[pl.ds(c * L, L)]],
                         rows_v, sem_g).wait()
        wrow = wb_v[pl.ds(c * L, L)]
        wsp = [_take16(wrow, jnp.full((L,), i, jnp.int32)) for i in range(L)]

        def qstep(q, carry):
            for t in range(TPC):
                r0 = rows_v[2 * t, pl.ds(q * L, L)]
                r1 = rows_v[2 * t + 1, pl.ds(q * L, L)]
                outb_v[t, pl.ds(q * L, L)] = (wsp[2 * t] * r0
                                              + wsp[2 * t + 1] * r1)
            return carry

        lax.fori_loop(0, HIDDEN // L, qstep, 0)
        row0 = wid * (T // NW) + c * TPC
        pltpu.sync_copy(outb_v, out_hbm.at[pl.ds(row0, TPC)])


def _combine(ys, pos, w_pair):
    kfn = pl.kernel(
        _combine_body,
        out_type=jax.ShapeDtypeStruct((T, HIDDEN), jnp.float32),
        mesh=plsc.VectorSubcoreMesh(core_axis_name="c", subcore_axis_name="s"),
        scratch_types=(
            pltpu.VMEM((CPW,), jnp.int32),
            pltpu.VMEM((CPW,), jnp.float32),
            pltpu.VMEM((2 * TPC, HIDDEN), jnp.float32),
            pltpu.VMEM((TPC, HIDDEN), jnp.float32),
            pltpu.SemaphoreType.DMA,
        ),
    )
    return kfn(ys, pos, w_pair)


@jax.jit
def _run(x, W_router, w1, w2, w3):
    B, S, H = x.shape
    xf = x.reshape(T, H)
    wr = jnp.zeros((EP, H), x.dtype).at[:E].set(W_router)
    idx_out, w_out, base, bexp, bnr = _router(xf, wr)
    e_pair = idx_out[:, :2].reshape(NPAIR)
    w_pair = w_out[:, :2].reshape(NPAIR)
    base16 = base[:, :L]
    xs, pos = _dispatch(e_pair, base16, xf)
    ys = _gmm(bexp[:, 0], bnr[:, 0], xs, w1, w3, w2)
    out = _combine(ys, pos, w_pair)
    return out.reshape(B, S, H)


def kernel(x, W_router, w1, w2, w3):
    return _run(x, W_router, w1, w2, w3)


# speedup vs baseline: 1.2842x; 1.2842x over previous
import functools

import jax
import jax.numpy as jnp
from jax import lax
from jax.experimental import pallas as pl
from jax.experimental.pallas import tpu as pltpu
from jax.experimental.pallas import tpu_sc as plsc

HIDDEN = 1024
FFN = 4096
E = 8
EP = 128
T = 2048
NPAIR = 2 * T
BM = 576
NB = -(-(NPAIR + E * (BM - 1)) // BM)
P = NB * BM
BF = 512
NF = FFN // BF
NW = 32
CPW = NPAIR // NW
L = 16
NG = NPAIR // L


def _router_body(x_ref, wr_ref, idx_ref, w_ref, base_ref, bexp_ref, bnr_ref):
    x = x_ref[...]
    wr = wr_ref[...]
    logits = lax.dot_general(x, wr, (((1,), (1,)), ((), ())),
                             preferred_element_type=jnp.float32)
    col = lax.broadcasted_iota(jnp.int32, logits.shape, 1)
    neg = jnp.float32(-1e30)
    logits = jnp.where(col < E, logits, neg)
    m1 = jnp.max(logits, axis=1, keepdims=True)
    a1 = jnp.min(jnp.where(logits == m1, col, EP), axis=1, keepdims=True)
    l2 = jnp.where(col == a1, neg, logits)
    m2 = jnp.max(l2, axis=1, keepdims=True)
    a2 = jnp.min(jnp.where(l2 == m2, col, EP), axis=1, keepdims=True)
    wtop = 1.0 / (1.0 + jnp.exp(m2 - m1))
    idx_ref[...] = jnp.where(col == 0, a1, 0) + jnp.where(col == 1, a2, 0)
    w_ref[...] = (jnp.where(col == 0, wtop, 0.0)
                  + jnp.where(col == 1, 1.0 - wtop, 0.0))

    onehot = (jnp.where(col == a1, 1.0, 0.0)
              + jnp.where(col == a2, 1.0, 0.0))
    trow = lax.broadcasted_iota(jnp.int32, (NG, T), 1)
    wrow = lax.broadcasted_iota(jnp.int32, (NG, T), 0)
    seg = jnp.where(trow // (T // NG) == wrow, 1.0, 0.0)
    hist = jnp.dot(seg, onehot, preferred_element_type=jnp.float32)
    total = jnp.sum(hist, axis=0, keepdims=True)
    padded = jnp.floor((total + (BM - 1)) / BM) * BM
    li = lax.broadcasted_iota(jnp.int32, (EP, EP), 0)
    lj = lax.broadcasted_iota(jnp.int32, (EP, EP), 1)
    lower = jnp.where(li < lj, 1.0, 0.0)
    gstart = jnp.dot(padded, lower, preferred_element_type=jnp.float32)
    wi = lax.broadcasted_iota(jnp.int32, (NG, NG), 0)
    wj = lax.broadcasted_iota(jnp.int32, (NG, NG), 1)
    wlow = jnp.where(wj < wi, 1.0, 0.0)
    prefix = jnp.dot(wlow, hist, preferred_element_type=jnp.float32)
    base_ref[...] = (gstart + prefix).astype(jnp.int32)

    bs = lax.broadcasted_iota(jnp.int32, (NB, EP), 0).astype(jnp.float32) * BM
    ecol = lax.broadcasted_iota(jnp.int32, (NB, EP), 1).astype(jnp.float32)
    gs = jnp.broadcast_to(gstart, (NB, EP))
    pd = jnp.broadcast_to(padded, (NB, EP))
    tt = jnp.broadcast_to(total, (NB, EP))
    inb = jnp.where((bs >= gs) & (bs < gs + pd), 1.0, 0.0)
    anyb = jnp.sum(inb, axis=1, keepdims=True)
    laste = jnp.max(jnp.where(total > 0, ecol[:1], 0.0), axis=1, keepdims=True)
    bexp = jnp.sum(inb * ecol, axis=1, keepdims=True) + (1.0 - anyb) * laste
    nr = jnp.clip(tt - (bs - gs), 0.0, float(BM))
    bnr = jnp.sum(inb * nr, axis=1, keepdims=True)
    ccol = lax.broadcasted_iota(jnp.int32, (NB, EP), 1)
    bexp_ref[...] = jnp.where(ccol == 0, bexp.astype(jnp.int32), 0)
    bnr_ref[...] = jnp.where(ccol == 0, bnr.astype(jnp.int32), 0)


def _router(xf, wr):
    return pl.pallas_call(
        _router_body,
        out_shape=(
            jax.ShapeDtypeStruct((T, EP), jnp.int32),
            jax.ShapeDtypeStruct((T, EP), jnp.float32),
            jax.ShapeDtypeStruct((NG, EP), jnp.int32),
            jax.ShapeDtypeStruct((NB, EP), jnp.int32),
            jax.ShapeDtypeStruct((NB, EP), jnp.int32),
        ),
    )(xf, wr)


def _iota16():
    return lax.iota(jnp.int32, L)


def _take16(v, idx):
    return lax.gather(
        v, idx[:, None],
        lax.GatherDimensionNumbers(offset_dims=(), collapsed_slice_dims=(0,),
                                   start_index_map=(0,)),
        slice_sizes=(1,),
        mode=lax.GatherScatterMode.PROMISE_IN_BOUNDS)


def _dispatch_body(epair_hbm, base_hbm, x_hbm, xs_hbm, pos_hbm,
                   eids_v, base_v, posb_v, tokb_v, rows_v, sem_g):
    wid = lax.axis_index("s") * 2 + lax.axis_index("c")
    p0 = wid * CPW
    g0 = pl.multiple_of(p0 // L, CPW // L)
    pltpu.sync_copy(epair_hbm.at[pl.ds(p0, CPW)], eids_v)
    pltpu.sync_copy(base_hbm.at[pl.ds(g0, CPW // L)], base_v)
    iot = _iota16()
    for j in range(CPW // L):
        base_j = base_v[j]
        v = eids_v[pl.ds(j * L, L)]
        bp = _take16(base_j, v)
        rank = jnp.zeros((L,), jnp.int32)
        for k in range(1, L):
            vk = _take16(v, jnp.maximum(iot - k, 0))
            rank = rank + jnp.where((iot >= k) & (vk == v), 1, 0)
        posb_v[j // 4, pl.ds((j % 4) * L, L)] = bp + rank
        tokb_v[j // 4, pl.ds((j % 4) * L, L)] = (p0 + j * L + iot) >> 1
    for c in range(2):
        pltpu.sync_copy(posb_v.at[c], pos_hbm.at[pl.ds(p0 + c * 64, 64)])
        pltpu.async_copy(x_hbm.at[tokb_v.at[c]], rows_v, sem_g).wait()
        pltpu.sync_copy(rows_v, xs_hbm.at[posb_v.at[c]])


def _dispatch(e_pair, base, xf):
    kfn = pl.kernel(
        _dispatch_body,
        out_type=(
            jax.ShapeDtypeStruct((P, HIDDEN), jnp.float32),
            jax.ShapeDtypeStruct((NPAIR,), jnp.int32),
        ),
        mesh=plsc.VectorSubcoreMesh(core_axis_name="c", subcore_axis_name="s"),
        scratch_types=(
            pltpu.VMEM((CPW,), jnp.int32),
            pltpu.VMEM((CPW // L, L), jnp.int32),
            pltpu.VMEM((2, 64), jnp.int32),
            pltpu.VMEM((2, 64), jnp.int32),
            pltpu.VMEM((64, HIDDEN), jnp.float32),
            pltpu.SemaphoreType.DMA,
        ),
    )
    return kfn(e_pair, base, xf)


def _gmm_body(bexp_ref, bnr_ref, xs_ref, w1_ref, w3_ref, w2_ref, ys_ref):
    b = pl.program_id(0)
    f = pl.program_id(1)

    @pl.when(bnr_ref[b] > 0)
    def _():
        x = xs_ref[...]
        h1 = jnp.dot(x, w1_ref[0], preferred_element_type=jnp.float32)
        h3 = jnp.dot(x, w3_ref[0], preferred_element_type=jnp.float32)
        h = (h1 * jax.nn.sigmoid(h1)) * h3
        delta = jnp.dot(h, w2_ref[0], preferred_element_type=jnp.float32)

        @pl.when(f == 0)
        def _():
            ys_ref[...] = delta

        @pl.when(f != 0)
        def _():
            ys_ref[...] += delta


def _gmm(bexp, bnr, xs, w1, w3, w2):
    grid_spec = pltpu.PrefetchScalarGridSpec(
        num_scalar_prefetch=2,
        grid=(NB, NF),
        in_specs=[
            pl.BlockSpec((BM, HIDDEN),
                         lambda b, f, be, bn: (jnp.where(bn[b] > 0, b, 0), 0)),
            pl.BlockSpec((1, HIDDEN, BF), lambda b, f, be, bn: (be[b], 0, f)),
            pl.BlockSpec((1, HIDDEN, BF), lambda b, f, be, bn: (be[b], 0, f)),
            pl.BlockSpec((1, BF, HIDDEN), lambda b, f, be, bn: (be[b], f, 0)),
        ],
        out_specs=pl.BlockSpec((BM, HIDDEN), lambda b, f, be, bn: (b, 0)),
    )
    return pl.pallas_call(
        _gmm_body,
        grid_spec=grid_spec,
        out_shape=jax.ShapeDtypeStruct((P, HIDDEN), jnp.float32),
        compiler_params=pltpu.CompilerParams(
            dimension_semantics=("arbitrary", "arbitrary"),
        ),
    )(bexp, bnr, xs, w1, w3, w2)


TPC = 8


def _combine_body(ys_hbm, pos_hbm, wp_hbm, out_hbm,
                  posb_v, wb_v, rows_v, outb_v, sem_g):
    wid = lax.axis_index("s") * 2 + lax.axis_index("c")
    p0 = wid * CPW
    pltpu.sync_copy(pos_hbm.at[pl.ds(p0, CPW)], posb_v)
    pltpu.sync_copy(wp_hbm.at[pl.ds(p0, CPW)], wb_v)
    nch = CPW // (2 * TPC)
    for c in range(nch):
        pltpu.async_copy(ys_hbm.at[posb_v.at[pl.ds(c * L, L)]],
                         rows_v, sem_g).wait()
        wrow = wb_v[pl.ds(c * L, L)]
        wsp = [_take16(wrow, jnp.full((L,), i, jnp.int32)) for i in range(L)]

        def qstep(q, carry):
            for t in range(TPC):
                r0 = rows_v[2 * t, pl.ds(q * L, L)]
                r1 = rows_v[2 * t + 1, pl.ds(q * L, L)]
                outb_v[t, pl.ds(q * L, L)] = (wsp[2 * t] * r0
                                              + wsp[2 * t + 1] * r1)
            return carry

        lax.fori_loop(0, HIDDEN // L, qstep, 0)
        row0 = wid * (T // NW) + c * TPC
        pltpu.sync_copy(outb_v, out_hbm.at[pl.ds(row0, TPC)])


def _combine(ys, pos, w_pair):
    kfn = pl.kernel(
        _combine_body,
        out_type=jax.ShapeDtypeStruct((T, HIDDEN), jnp.float32),
        mesh=plsc.VectorSubcoreMesh(core_axis_name="c", subcore_axis_name="s"),
        scratch_types=(
            pltpu.VMEM((CPW,), jnp.int32),
            pltpu.VMEM((CPW,), jnp.float32),
            pltpu.VMEM((2 * TPC, HIDDEN), jnp.float32),
            pltpu.VMEM((TPC, HIDDEN), jnp.float32),
            pltpu.SemaphoreType.DMA,
        ),
    )
    return kfn(ys, pos, w_pair)


@jax.jit
def _run(x, W_router, w1, w2, w3):
    B, S, H = x.shape
    xf = x.reshape(T, H)
    wr = jnp.zeros((EP, H), x.dtype).at[:E].set(W_router)
    idx_out, w_out, base, bexp, bnr = _router(xf, wr)
    e_pair = idx_out[:, :2].reshape(NPAIR)
    w_pair = w_out[:, :2].reshape(NPAIR)
    base16 = base[:, :L]
    xs, pos = _dispatch(e_pair, base16, xf)
    ys = _gmm(bexp[:, 0], bnr[:, 0], xs, w1, w3, w2)
    out = _combine(ys, pos, w_pair)
    return out.reshape(B, S, H)


def kernel(x, W_router, w1, w2, w3):
    return _run(x, W_router, w1, w2, w3)


# speedup vs baseline: 1.3037x; 1.0152x over previous
import functools

import jax
import jax.numpy as jnp
from jax import lax
from jax.experimental import pallas as pl
from jax.experimental.pallas import tpu as pltpu
from jax.experimental.pallas import tpu_sc as plsc

HIDDEN = 1024
FFN = 4096
E = 8
EP = 128
T = 2048
NPAIR = 2 * T
BM = 576
NB = -(-(NPAIR + E * (BM - 1)) // BM)
P = NB * BM
BF = 512
NF = FFN // BF
NW = 32
CPW = NPAIR // NW
L = 16
NG = NPAIR // L


def _router_body(x_ref, wr_ref, idx_ref, w_ref, base_ref, bexp_ref, bnr_ref):
    x = x_ref[...]
    wr = wr_ref[...]
    logits = lax.dot_general(x, wr, (((1,), (1,)), ((), ())),
                             preferred_element_type=jnp.float32)
    col = lax.broadcasted_iota(jnp.int32, logits.shape, 1)
    neg = jnp.float32(-1e30)
    logits = jnp.where(col < E, logits, neg)
    m1 = jnp.max(logits, axis=1, keepdims=True)
    a1 = jnp.min(jnp.where(logits == m1, col, EP), axis=1, keepdims=True)
    l2 = jnp.where(col == a1, neg, logits)
    m2 = jnp.max(l2, axis=1, keepdims=True)
    a2 = jnp.min(jnp.where(l2 == m2, col, EP), axis=1, keepdims=True)
    wtop = 1.0 / (1.0 + jnp.exp(m2 - m1))
    idx_ref[...] = jnp.where(col == 0, a1, 0) + jnp.where(col == 1, a2, 0)
    w_ref[...] = (jnp.where(col == 0, wtop, 0.0)
                  + jnp.where(col == 1, 1.0 - wtop, 0.0))

    onehot = (jnp.where(col == a1, 1.0, 0.0)
              + jnp.where(col == a2, 1.0, 0.0))
    trow = lax.broadcasted_iota(jnp.int32, (NG, T), 1)
    wrow = lax.broadcasted_iota(jnp.int32, (NG, T), 0)
    seg = jnp.where(trow // (T // NG) == wrow, 1.0, 0.0)
    hist = jnp.dot(seg, onehot, preferred_element_type=jnp.float32)
    total = jnp.sum(hist, axis=0, keepdims=True)
    padded = jnp.floor((total + (BM - 1)) / BM) * BM
    li = lax.broadcasted_iota(jnp.int32, (EP, EP), 0)
    lj = lax.broadcasted_iota(jnp.int32, (EP, EP), 1)
    lower = jnp.where(li < lj, 1.0, 0.0)
    gstart = jnp.dot(padded, lower, preferred_element_type=jnp.float32)
    wi = lax.broadcasted_iota(jnp.int32, (NG, NG), 0)
    wj = lax.broadcasted_iota(jnp.int32, (NG, NG), 1)
    wlow = jnp.where(wj < wi, 1.0, 0.0)
    prefix = jnp.dot(wlow, hist, preferred_element_type=jnp.float32)
    base_ref[...] = (gstart + prefix).astype(jnp.int32)

    bs = lax.broadcasted_iota(jnp.int32, (NB, EP), 0).astype(jnp.float32) * BM
    ecol = lax.broadcasted_iota(jnp.int32, (NB, EP), 1).astype(jnp.float32)
    gs = jnp.broadcast_to(gstart, (NB, EP))
    pd = jnp.broadcast_to(padded, (NB, EP))
    tt = jnp.broadcast_to(total, (NB, EP))
    inb = jnp.where((bs >= gs) & (bs < gs + pd), 1.0, 0.0)
    anyb = jnp.sum(inb, axis=1, keepdims=True)
    laste = jnp.max(jnp.where(total > 0, ecol[:1], 0.0), axis=1, keepdims=True)
    bexp = jnp.sum(inb * ecol, axis=1, keepdims=True) + (1.0 - anyb) * laste
    nr = jnp.clip(tt - (bs - gs), 0.0, float(BM))
    bnr = jnp.sum(inb * nr, axis=1, keepdims=True)
    ccol = lax.broadcasted_iota(jnp.int32, (NB, EP), 1)
    lastb = jnp.sum(anyb, axis=0, keepdims=True) - 1.0
    bexp_ref[...] = (jnp.where(ccol == 0, bexp.astype(jnp.int32), 0)
                     + jnp.where(ccol == 1, lastb.astype(jnp.int32), 0))
    bnr_ref[...] = jnp.where(ccol == 0, bnr.astype(jnp.int32), 0)


def _router(xf, wr):
    return pl.pallas_call(
        _router_body,
        out_shape=(
            jax.ShapeDtypeStruct((T, EP), jnp.int32),
            jax.ShapeDtypeStruct((T, EP), jnp.float32),
            jax.ShapeDtypeStruct((NG, EP), jnp.int32),
            jax.ShapeDtypeStruct((NB, EP), jnp.int32),
            jax.ShapeDtypeStruct((NB, EP), jnp.int32),
        ),
    )(xf, wr)


def _iota16():
    return lax.iota(jnp.int32, L)


def _take16(v, idx):
    return lax.gather(
        v, idx[:, None],
        lax.GatherDimensionNumbers(offset_dims=(), collapsed_slice_dims=(0,),
                                   start_index_map=(0,)),
        slice_sizes=(1,),
        mode=lax.GatherScatterMode.PROMISE_IN_BOUNDS)


def _dispatch_body(epair_hbm, base_hbm, x_hbm, xs_hbm, pos_hbm,
                   eids_v, base_v, posb_v, tokb_v, rows_v, sem_g):
    wid = lax.axis_index("s") * 2 + lax.axis_index("c")
    p0 = wid * CPW
    g0 = pl.multiple_of(p0 // L, CPW // L)
    pltpu.sync_copy(epair_hbm.at[pl.ds(p0, CPW)], eids_v)
    pltpu.sync_copy(base_hbm.at[pl.ds(g0, CPW // L)], base_v)
    iot = _iota16()
    for j in range(CPW // L):
        base_j = base_v[j]
        v = eids_v[pl.ds(j * L, L)]
        bp = _take16(base_j, v)
        rank = jnp.zeros((L,), jnp.int32)
        for k in range(1, L):
            vk = _take16(v, jnp.maximum(iot - k, 0))
            rank = rank + jnp.where((iot >= k) & (vk == v), 1, 0)
        posb_v[j // 4, pl.ds((j % 4) * L, L)] = bp + rank
        tokb_v[j // 4, pl.ds((j % 4) * L, L)] = (p0 + j * L + iot) >> 1
    for c in range(2):
        pltpu.sync_copy(posb_v.at[c], pos_hbm.at[pl.ds(p0 + c * 64, 64)])
        pltpu.async_copy(x_hbm.at[tokb_v.at[c]], rows_v, sem_g).wait()
        pltpu.sync_copy(rows_v, xs_hbm.at[posb_v.at[c]])


def _dispatch(e_pair, base, xf):
    kfn = pl.kernel(
        _dispatch_body,
        out_type=(
            jax.ShapeDtypeStruct((P, HIDDEN), jnp.float32),
            jax.ShapeDtypeStruct((NPAIR,), jnp.int32),
        ),
        mesh=plsc.VectorSubcoreMesh(core_axis_name="c", subcore_axis_name="s"),
        scratch_types=(
            pltpu.VMEM((CPW,), jnp.int32),
            pltpu.VMEM((CPW // L, L), jnp.int32),
            pltpu.VMEM((2, 64), jnp.int32),
            pltpu.VMEM((2, 64), jnp.int32),
            pltpu.VMEM((64, HIDDEN), jnp.float32),
            pltpu.SemaphoreType.DMA,
        ),
    )
    return kfn(e_pair, base, xf)


def _gmm_body(bexp_ref, bnr_ref, lb_ref, xs_ref, w1_ref, w3_ref, w2_ref,
              ys_ref):
    b = pl.program_id(0)
    f = pl.program_id(1)

    @pl.when(bnr_ref[b] > 0)
    def _():
        x = xs_ref[...]
        h1 = jnp.dot(x, w1_ref[0], preferred_element_type=jnp.float32)
        h3 = jnp.dot(x, w3_ref[0], preferred_element_type=jnp.float32)
        h = (h1 * jax.nn.sigmoid(h1)) * h3
        delta = jnp.dot(h, w2_ref[0], preferred_element_type=jnp.float32)

        @pl.when(f == 0)
        def _():
            ys_ref[...] = delta

        @pl.when(f != 0)
        def _():
            ys_ref[...] += delta


def _gmm(bexp, bnr, lastb, xs, w1, w3, w2):
    grid_spec = pltpu.PrefetchScalarGridSpec(
        num_scalar_prefetch=3,
        grid=(NB, NF),
        in_specs=[
            pl.BlockSpec(
                (BM, HIDDEN),
                lambda b, f, be, bn, lb: (jnp.where(bn[b] > 0, b, lb[0]), 0)),
            pl.BlockSpec((1, HIDDEN, BF),
                         lambda b, f, be, bn, lb: (be[b], 0, f)),
            pl.BlockSpec((1, HIDDEN, BF),
                         lambda b, f, be, bn, lb: (be[b], 0, f)),
            pl.BlockSpec((1, BF, HIDDEN),
                         lambda b, f, be, bn, lb: (be[b], f, 0)),
        ],
        out_specs=pl.BlockSpec(
            (BM, HIDDEN),
            lambda b, f, be, bn, lb: (jnp.where(bn[b] > 0, b, lb[0]), 0)),
    )
    return pl.pallas_call(
        _gmm_body,
        grid_spec=grid_spec,
        out_shape=jax.ShapeDtypeStruct((P, HIDDEN), jnp.float32),
        compiler_params=pltpu.CompilerParams(
            dimension_semantics=("arbitrary", "arbitrary"),
        ),
    )(bexp, bnr, lastb, xs, w1, w3, w2)


TPC = 8


def _combine_body(ys_hbm, pos_hbm, wp_hbm, out_hbm,
                  posb_v, wb_v, rows_v, outb_v, sem_g):
    wid = lax.axis_index("s") * 2 + lax.axis_index("c")
    p0 = wid * CPW
    pltpu.sync_copy(pos_hbm.at[pl.ds(p0, CPW)], posb_v)
    pltpu.sync_copy(wp_hbm.at[pl.ds(p0, CPW)], wb_v)
    nch = CPW // (2 * TPC)
    for c in range(nch):
        pltpu.async_copy(ys_hbm.at[posb_v.at[pl.ds(c * L, L)]],
                         rows_v, sem_g).wait()
        wrow = wb_v[pl.ds(c * L, L)]
        wsp = [_take16(wrow, jnp.full((L,), i, jnp.int32)) for i in range(L)]

        def qstep(q, carry):
            for t in range(TPC):
                r0 = rows_v[2 * t, pl.ds(q * L, L)]
                r1 = rows_v[2 * t + 1, pl.ds(q * L, L)]
                outb_v[t, pl.ds(q * L, L)] = (wsp[2 * t] * r0
                                              + wsp[2 * t + 1] * r1)
            return carry

        lax.fori_loop(0, HIDDEN // L, qstep, 0)
        row0 = wid * (T // NW) + c * TPC
        pltpu.sync_copy(outb_v, out_hbm.at[pl.ds(row0, TPC)])


def _combine(ys, pos, w_pair):
    kfn = pl.kernel(
        _combine_body,
        out_type=jax.ShapeDtypeStruct((T, HIDDEN), jnp.float32),
        mesh=plsc.VectorSubcoreMesh(core_axis_name="c", subcore_axis_name="s"),
        scratch_types=(
            pltpu.VMEM((CPW,), jnp.int32),
            pltpu.VMEM((CPW,), jnp.float32),
            pltpu.VMEM((2 * TPC, HIDDEN), jnp.float32),
            pltpu.VMEM((TPC, HIDDEN), jnp.float32),
            pltpu.SemaphoreType.DMA,
        ),
    )
    return kfn(ys, pos, w_pair)


@jax.jit
def _run(x, W_router, w1, w2, w3):
    B, S, H = x.shape
    xf = x.reshape(T, H)
    wr = jnp.zeros((EP, H), x.dtype).at[:E].set(W_router)
    idx_out, w_out, base, bexp, bnr = _router(xf, wr)
    e_pair = idx_out[:, :2].reshape(NPAIR)
    w_pair = w_out[:, :2].reshape(NPAIR)
    base16 = base[:, :L]
    xs, pos = _dispatch(e_pair, base16, xf)
    ys = _gmm(bexp[:, 0], bnr[:, 0], bexp[:, 1], xs, w1, w3, w2)
    out = _combine(ys, pos, w_pair)
    return out.reshape(B, S, H)


def kernel(x, W_router, w1, w2, w3):
    return _run(x, W_router, w1, w2, w3)


# speedup vs baseline: 1.3912x; 1.0671x over previous
import functools

import jax
import jax.numpy as jnp
from jax import lax
from jax.experimental import pallas as pl
from jax.experimental.pallas import tpu as pltpu
from jax.experimental.pallas import tpu_sc as plsc

HIDDEN = 1024
FFN = 4096
E = 8
EP = 128
T = 2048
NPAIR = 2 * T
BM = 576
NB = -(-(NPAIR + E * (BM - 1)) // BM)
P = NB * BM
BF = 1024
NF = FFN // BF
NW = 32
CPW = NPAIR // NW
L = 16
NG = NPAIR // L


def _router_body(x_ref, wr_ref, idx_ref, w_ref, base_ref, bexp_ref, bnr_ref):
    x = x_ref[...]
    wr = wr_ref[...]
    logits = lax.dot_general(x, wr, (((1,), (1,)), ((), ())),
                             preferred_element_type=jnp.float32)
    col = lax.broadcasted_iota(jnp.int32, logits.shape, 1)
    neg = jnp.float32(-1e30)
    logits = jnp.where(col < E, logits, neg)
    m1 = jnp.max(logits, axis=1, keepdims=True)
    a1 = jnp.min(jnp.where(logits == m1, col, EP), axis=1, keepdims=True)
    l2 = jnp.where(col == a1, neg, logits)
    m2 = jnp.max(l2, axis=1, keepdims=True)
    a2 = jnp.min(jnp.where(l2 == m2, col, EP), axis=1, keepdims=True)
    wtop = 1.0 / (1.0 + jnp.exp(m2 - m1))
    idx_ref[...] = jnp.where(col == 0, a1, 0) + jnp.where(col == 1, a2, 0)
    w_ref[...] = (jnp.where(col == 0, wtop, 0.0)
                  + jnp.where(col == 1, 1.0 - wtop, 0.0))

    onehot = (jnp.where(col == a1, 1.0, 0.0)
              + jnp.where(col == a2, 1.0, 0.0))
    trow = lax.broadcasted_iota(jnp.int32, (NG, T), 1)
    wrow = lax.broadcasted_iota(jnp.int32, (NG, T), 0)
    seg = jnp.where(trow // (T // NG) == wrow, 1.0, 0.0)
    hist = jnp.dot(seg, onehot, preferred_element_type=jnp.float32)
    total = jnp.sum(hist, axis=0, keepdims=True)
    padded = jnp.floor((total + (BM - 1)) / BM) * BM
    li = lax.broadcasted_iota(jnp.int32, (EP, EP), 0)
    lj = lax.broadcasted_iota(jnp.int32, (EP, EP), 1)
    lower = jnp.where(li < lj, 1.0, 0.0)
    gstart = jnp.dot(padded, lower, preferred_element_type=jnp.float32)
    wi = lax.broadcasted_iota(jnp.int32, (NG, NG), 0)
    wj = lax.broadcasted_iota(jnp.int32, (NG, NG), 1)
    wlow = jnp.where(wj < wi, 1.0, 0.0)
    prefix = jnp.dot(wlow, hist, preferred_element_type=jnp.float32)
    base_ref[...] = (gstart + prefix).astype(jnp.int32)

    bs = lax.broadcasted_iota(jnp.int32, (NB, EP), 0).astype(jnp.float32) * BM
    ecol = lax.broadcasted_iota(jnp.int32, (NB, EP), 1).astype(jnp.float32)
    gs = jnp.broadcast_to(gstart, (NB, EP))
    pd = jnp.broadcast_to(padded, (NB, EP))
    tt = jnp.broadcast_to(total, (NB, EP))
    inb = jnp.where((bs >= gs) & (bs < gs + pd), 1.0, 0.0)
    anyb = jnp.sum(inb, axis=1, keepdims=True)
    laste = jnp.max(jnp.where(total > 0, ecol[:1], 0.0), axis=1, keepdims=True)
    bexp = jnp.sum(inb * ecol, axis=1, keepdims=True) + (1.0 - anyb) * laste
    nr = jnp.clip(tt - (bs - gs), 0.0, float(BM))
    bnr = jnp.sum(inb * nr, axis=1, keepdims=True)
    ccol = lax.broadcasted_iota(jnp.int32, (NB, EP), 1)
    lastb = jnp.sum(anyb, axis=0, keepdims=True) - 1.0
    bexp_ref[...] = (jnp.where(ccol == 0, bexp.astype(jnp.int32), 0)
                     + jnp.where(ccol == 1, lastb.astype(jnp.int32), 0))
    bnr_ref[...] = jnp.where(ccol == 0, bnr.astype(jnp.int32), 0)


def _router(xf, wr):
    return pl.pallas_call(
        _router_body,
        out_shape=(
            jax.ShapeDtypeStruct((T, EP), jnp.int32),
            jax.ShapeDtypeStruct((T, EP), jnp.float32),
            jax.ShapeDtypeStruct((NG, EP), jnp.int32),
            jax.ShapeDtypeStruct((NB, EP), jnp.int32),
            jax.ShapeDtypeStruct((NB, EP), jnp.int32),
        ),
    )(xf, wr)


def _iota16():
    return lax.iota(jnp.int32, L)


def _take16(v, idx):
    return lax.gather(
        v, idx[:, None],
        lax.GatherDimensionNumbers(offset_dims=(), collapsed_slice_dims=(0,),
                                   start_index_map=(0,)),
        slice_sizes=(1,),
        mode=lax.GatherScatterMode.PROMISE_IN_BOUNDS)


def _dispatch_body(epair_hbm, base_hbm, x_hbm, xs_hbm, pos_hbm,
                   eids_v, base_v, posb_v, tokb_v, rows_v, sem_g):
    wid = lax.axis_index("s") * 2 + lax.axis_index("c")
    p0 = wid * CPW
    g0 = pl.multiple_of(p0 // L, CPW // L)
    pltpu.sync_copy(epair_hbm.at[pl.ds(p0, CPW)], eids_v)
    pltpu.sync_copy(base_hbm.at[pl.ds(g0, CPW // L)], base_v)
    iot = _iota16()
    for j in range(CPW // L):
        base_j = base_v[j]
        v = eids_v[pl.ds(j * L, L)]
        bp = _take16(base_j, v)
        rank = jnp.zeros((L,), jnp.int32)
        for k in range(1, L):
            vk = _take16(v, jnp.maximum(iot - k, 0))
            rank = rank + jnp.where((iot >= k) & (vk == v), 1, 0)
        posb_v[j // 4, pl.ds((j % 4) * L, L)] = bp + rank
        tokb_v[j // 4, pl.ds((j % 4) * L, L)] = (p0 + j * L + iot) >> 1
    for c in range(2):
        pltpu.sync_copy(posb_v.at[c], pos_hbm.at[pl.ds(p0 + c * 64, 64)])
        pltpu.async_copy(x_hbm.at[tokb_v.at[c]], rows_v, sem_g).wait()
        pltpu.sync_copy(rows_v, xs_hbm.at[posb_v.at[c]])


def _dispatch(e_pair, base, xf):
    kfn = pl.kernel(
        _dispatch_body,
        out_type=(
            jax.ShapeDtypeStruct((P, HIDDEN), jnp.float32),
            jax.ShapeDtypeStruct((NPAIR,), jnp.int32),
        ),
        mesh=plsc.VectorSubcoreMesh(core_axis_name="c", subcore_axis_name="s"),
        scratch_types=(
            pltpu.VMEM((CPW,), jnp.int32),
            pltpu.VMEM((CPW // L, L), jnp.int32),
            pltpu.VMEM((2, 64), jnp.int32),
            pltpu.VMEM((2, 64), jnp.int32),
            pltpu.VMEM((64, HIDDEN), jnp.float32),
            pltpu.SemaphoreType.DMA,
        ),
    )
    return kfn(e_pair, base, xf)


def _gmm_body(bexp_ref, bnr_ref, lb_ref, xs_ref, w1_ref, w3_ref, w2_ref,
              ys_ref):
    b = pl.program_id(0)
    f = pl.program_id(1)

    @pl.when(bnr_ref[b] > 0)
    def _():
        x = xs_ref[...]
        h1 = jnp.dot(x, w1_ref[0], preferred_element_type=jnp.float32)
        h3 = jnp.dot(x, w3_ref[0], preferred_element_type=jnp.float32)
        h = (h1 * jax.nn.sigmoid(h1)) * h3
        delta = jnp.dot(h, w2_ref[0], preferred_element_type=jnp.float32)

        @pl.when(f == 0)
        def _():
            ys_ref[...] = delta

        @pl.when(f != 0)
        def _():
            ys_ref[...] += delta


def _gmm(bexp, bnr, lastb, xs, w1, w3, w2):
    grid_spec = pltpu.PrefetchScalarGridSpec(
        num_scalar_prefetch=3,
        grid=(NB, NF),
        in_specs=[
            pl.BlockSpec(
                (BM, HIDDEN),
                lambda b, f, be, bn, lb: (jnp.where(bn[b] > 0, b, lb[0]), 0)),
            pl.BlockSpec((1, HIDDEN, BF),
                         lambda b, f, be, bn, lb: (be[b], 0, f)),
            pl.BlockSpec((1, HIDDEN, BF),
                         lambda b, f, be, bn, lb: (be[b], 0, f)),
            pl.BlockSpec((1, BF, HIDDEN),
                         lambda b, f, be, bn, lb: (be[b], f, 0)),
        ],
        out_specs=pl.BlockSpec(
            (BM, HIDDEN),
            lambda b, f, be, bn, lb: (jnp.where(bn[b] > 0, b, lb[0]), 0)),
    )
    return pl.pallas_call(
        _gmm_body,
        grid_spec=grid_spec,
        out_shape=jax.ShapeDtypeStruct((P, HIDDEN), jnp.float32),
        compiler_params=pltpu.CompilerParams(
            dimension_semantics=("arbitrary", "arbitrary"),
        ),
    )(bexp, bnr, lastb, xs, w1, w3, w2)


TPC = 8


def _combine_body(ys_hbm, pos_hbm, wp_hbm, out_hbm,
                  posb_v, wb_v, rows_v, outb_v, sem_g):
    wid = lax.axis_index("s") * 2 + lax.axis_index("c")
    p0 = wid * CPW
    pltpu.sync_copy(pos_hbm.at[pl.ds(p0, CPW)], posb_v)
    pltpu.sync_copy(wp_hbm.at[pl.ds(p0, CPW)], wb_v)
    nch = CPW // (2 * TPC)
    for c in range(nch):
        pltpu.async_copy(ys_hbm.at[posb_v.at[pl.ds(c * L, L)]],
                         rows_v, sem_g).wait()
        wrow = wb_v[pl.ds(c * L, L)]
        wsp = [_take16(wrow, jnp.full((L,), i, jnp.int32)) for i in range(L)]

        def qstep(q, carry):
            for t in range(TPC):
                r0 = rows_v[2 * t, pl.ds(q * L, L)]
                r1 = rows_v[2 * t + 1, pl.ds(q * L, L)]
                outb_v[t, pl.ds(q * L, L)] = (wsp[2 * t] * r0
                                              + wsp[2 * t + 1] * r1)
            return carry

        lax.fori_loop(0, HIDDEN // L, qstep, 0)
        row0 = wid * (T // NW) + c * TPC
        pltpu.sync_copy(outb_v, out_hbm.at[pl.ds(row0, TPC)])


def _combine(ys, pos, w_pair):
    kfn = pl.kernel(
        _combine_body,
        out_type=jax.ShapeDtypeStruct((T, HIDDEN), jnp.float32),
        mesh=plsc.VectorSubcoreMesh(core_axis_name="c", subcore_axis_name="s"),
        scratch_types=(
            pltpu.VMEM((CPW,), jnp.int32),
            pltpu.VMEM((CPW,), jnp.float32),
            pltpu.VMEM((2 * TPC, HIDDEN), jnp.float32),
            pltpu.VMEM((TPC, HIDDEN), jnp.float32),
            pltpu.SemaphoreType.DMA,
        ),
    )
    return kfn(ys, pos, w_pair)


@jax.jit
def _run(x, W_router, w1, w2, w3):
    B, S, H = x.shape
    xf = x.reshape(T, H)
    wr = jnp.zeros((EP, H), x.dtype).at[:E].set(W_router)
    idx_out, w_out, base, bexp, bnr = _router(xf, wr)
    e_pair = idx_out[:, :2].reshape(NPAIR)
    w_pair = w_out[:, :2].reshape(NPAIR)
    base16 = base[:, :L]
    xs, pos = _dispatch(e_pair, base16, xf)
    ys = _gmm(bexp[:, 0], bnr[:, 0], bexp[:, 1], xs, w1, w3, w2)
    out = _combine(ys, pos, w_pair)
    return out.reshape(B, S, H)


def kernel(x, W_router, w1, w2, w3):
    return _run(x, W_router, w1, w2, w3)
